# baseline (speedup 1.0000x reference)
import functools

import jax
import jax.numpy as jnp
from jax import lax
from jax.experimental import pallas as pl
from jax.experimental.pallas import tpu as pltpu
from jax.experimental.pallas import tpu_sc as plsc

N_VOCAB = 100000
D_MODEL = 32
BATCH = 1024

_NC = 2
_NS = 16
_L = 16
_NW = _NC * _NS
_BPW = BATCH // _NW

_N_TILE = 2048
_NSTEP = (N_VOCAB + _N_TILE - 1) // _N_TILE


def _gather_combine_body(a_idx_hbm, b_idx_hbm, fiber_hbm, out_hbm,
                         idx_a, idx_b, rows_a, rows_b, sem_a, sem_b):
    wid = lax.axis_index("s") * _NC + lax.axis_index("c")
    base = wid * _BPW
    pltpu.sync_copy(a_idx_hbm.at[pl.ds(base, _BPW)], idx_a)
    pltpu.sync_copy(b_idx_hbm.at[pl.ds(base, _BPW)], idx_b)
    ca = pltpu.async_copy(fiber_hbm.at[idx_a], rows_a, sem_a)
    cb = pltpu.async_copy(fiber_hbm.at[idx_b], rows_b, sem_b)
    ca.wait()
    cb.wait()
    for i in range(_BPW):
        for j in range(D_MODEL // _L):
            sl = pl.ds(j * _L, _L)
            rows_a[i, sl] = rows_a[i, sl] + rows_b[i, sl]
    pltpu.sync_copy(rows_a, out_hbm.at[pl.ds(base, _BPW)])


_gather_combine = functools.partial(
    pl.kernel,
    out_type=jax.ShapeDtypeStruct((BATCH, D_MODEL), jnp.float32),
    mesh=plsc.VectorSubcoreMesh(core_axis_name="c", subcore_axis_name="s"),
    compiler_params=pltpu.CompilerParams(use_tc_tiling_on_sc=False),
    scratch_types=[
        pltpu.VMEM((_BPW,), jnp.int32),
        pltpu.VMEM((_BPW,), jnp.int32),
        pltpu.VMEM((_BPW, D_MODEL), jnp.float32),
        pltpu.VMEM((_BPW, D_MODEL), jnp.float32),
        pltpu.SemaphoreType.DMA,
        pltpu.SemaphoreType.DMA,
    ],
)(_gather_combine_body)


_NBUF = 3
_B_T = 32
_SUB = 4
_NG = BATCH // (_B_T * _SUB)
_NT = BATCH // _B_T


def _unembed_body(combo_ref, wt_ref, out_ref, acc_ref, sems):
    i = pl.program_id(0)
    for k in range(_SUB):
        t = i * _SUB + k
        buf = lax.rem(t, _NBUF)

        @pl.when(t >= _NBUF)
        def _wait_prev():
            pltpu.make_async_copy(
                acc_ref.at[buf],
                out_ref.at[pl.ds((t - _NBUF) * _B_T, _B_T)],
                sems.at[buf],
            ).wait()

        acc_ref[buf] = lax.dot_general(
            combo_ref[pl.ds(k * _B_T, _B_T), :], wt_ref[...],
            (((1,), (0,)), ((), ())),
            preferred_element_type=jnp.float32,
        )
        pltpu.make_async_copy(
            acc_ref.at[buf],
            out_ref.at[pl.ds(t * _B_T, _B_T)],
            sems.at[buf],
        ).start()

    @pl.when(i == _NG - 1)
    def _drain():
        for j in range(_NT - _NBUF, _NT):
            pltpu.make_async_copy(
                acc_ref.at[j % _NBUF],
                out_ref.at[pl.ds(j * _B_T, _B_T)],
                sems.at[j % _NBUF],
            ).wait()


def _unembed(combo, w_t):
    return pl.pallas_call(
        _unembed_body,
        grid=(_NG,),
        in_specs=[
            pl.BlockSpec((_B_T * _SUB, D_MODEL), lambda i: (i, 0)),
            pl.BlockSpec((D_MODEL, N_VOCAB), lambda i: (0, 0)),
        ],
        out_specs=pl.BlockSpec(memory_space=pl.ANY),
        out_shape=jax.ShapeDtypeStruct((BATCH, N_VOCAB), jnp.float32),
        scratch_shapes=[
            pltpu.VMEM((_NBUF, _B_T, N_VOCAB), jnp.float32),
            pltpu.SemaphoreType.DMA((_NBUF,)),
        ],
    )(combo, w_t)


@jax.jit
def kernel(a_idx, b_idx, fiber_weight, unembed_weight):
    combo = _gather_combine(a_idx, b_idx, fiber_weight)
    return _unembed(combo, unembed_weight.T)
